# baseline (speedup 1.0000x reference)
import functools

import jax
import jax.numpy as jnp
from jax import lax
from jax.experimental import pallas as pl
from jax.experimental.pallas import tpu as pltpu
from jax.experimental.pallas import tpu_sc as plsc

N_NODES = 10000
N_EDGES = 320000
LATENT = 128

NUM_CORES = 2
NUM_SUBCORES = 16
NUM_WORKERS = NUM_CORES * NUM_SUBCORES
CHUNK = 128
N_CHUNKS = 2560
E_PAD = N_CHUNKS * CHUNK
N_ROWS = 10240
SLAB = N_ROWS // NUM_SUBCORES
CHUNKS_PER_WORKER = N_CHUNKS // NUM_WORKERS
EDGES_PER_WORKER = CHUNKS_PER_WORKER * CHUNK

BM = 2560
BN = 2048


def _deg_body(dst_hbm, out_hbm, didx, ones_v, zrow, acc, sem):
    c = lax.axis_index("c")
    s = lax.axis_index("s")

    def zinit(j, _):
        zrow[pl.ds(j * 16, 16)] = jnp.zeros((16,), jnp.float32)
        return _

    lax.fori_loop(0, SLAB // 16, zinit, None)
    for j in range(CHUNK // 16):
        ones_v[pl.ds(j * 16, 16)] = jnp.ones((16,), jnp.float32)
    pltpu.sync_copy(zrow, acc.at[pl.ds(s * SLAB, SLAB)])
    plsc.subcore_barrier()

    w = c * NUM_SUBCORES + s
    pltpu.sync_copy(dst_hbm.at[pl.ds(w * CHUNKS_PER_WORKER, CHUNKS_PER_WORKER)],
                    didx)
    copies = [
        pltpu.async_copy(ones_v, acc.at[didx.at[i]], sem, add=True)
        for i in range(CHUNKS_PER_WORKER)
    ]
    for cp in copies:
        cp.wait()
    plsc.subcore_barrier()

    r0 = s * SLAB
    pltpu.sync_copy(acc.at[pl.ds(r0, SLAB)],
                    out_hbm.at[pl.ds(c * N_ROWS + r0, SLAB)])


def _sc_degree(dst2d):
    mesh = plsc.VectorSubcoreMesh(core_axis_name="c", subcore_axis_name="s")
    k = pl.kernel(
        _deg_body,
        mesh=mesh,
        out_type=jax.ShapeDtypeStruct((NUM_CORES * N_ROWS,), jnp.float32),
        scratch_types=[
            pltpu.VMEM((CHUNKS_PER_WORKER, CHUNK), jnp.int32),
            pltpu.VMEM((CHUNK,), jnp.float32),
            pltpu.VMEM((SLAB,), jnp.float32),
            pltpu.VMEM_SHARED((N_ROWS,), jnp.float32),
            pltpu.SemaphoreType.DMA,
        ],
    )
    return k(dst2d)


def _sc_scatter(xs_ext, src1d, dst1d):
    mesh = plsc.VectorSubcoreMesh(core_axis_name="c", subcore_axis_name="s")

    def body(xs_hbm, src_hbm, dst_hbm, out_hbm,
             sidx, db0, db1, db2, db3, r0b, r1b, acc,
             i0, i1, i2, i3, g0, g1):
        rows = (r0b, r1b)
        dbuf = (db0, db1, db2, db3)
        isem = (i0, i1, i2, i3)
        gsem = (g0, g1)
        c = lax.axis_index("c")
        s = lax.axis_index("s")

        def zinit(j, _):
            for j16 in range(LATENT // 16):
                r0b[j, pl.ds(j16 * 16, 16)] = jnp.zeros((16,), jnp.float32)
            return _

        lax.fori_loop(0, CHUNK, zinit, None)
        r0 = s * SLAB
        for t in range(SLAB // CHUNK):
            pltpu.sync_copy(r0b, acc.at[pl.ds(r0 + t * CHUNK, CHUNK)])
        plsc.subcore_barrier()

        w = c * NUM_SUBCORES + s
        e0 = w * EDGES_PER_WORKER
        pltpu.sync_copy(src_hbm.at[pl.ds(e0, EDGES_PER_WORKER)], sidx)

        def didx_fetch(ci, q):
            return pltpu.make_async_copy(
                dst_hbm.at[pl.ds(e0 + ci * CHUNK, CHUNK)], dbuf[q], isem[q])

        def gather_copy(ci, b):
            return pltpu.make_async_copy(
                xs_hbm.at[sidx.at[pl.ds(ci * CHUNK, CHUNK)]], rows[b],
                gsem[b])

        didx_fetch(0, 0).start()
        didx_fetch(1, 1).start()
        gather_copy(0, 0).start()
        gather_copy(1, 1).start()

        def group(g, _):
            for j in range(4):
                ci = 4 * g + j
                b = j % 2
                q = j
                ni = ci + 2
                nq = (j + 2) % 4

                @pl.when(ni < CHUNKS_PER_WORKER)
                def _():
                    didx_fetch(ni, nq).start()

                gather_copy(ci, b).wait()
                didx_fetch(ci, q).wait()
                pltpu.sync_copy(rows[b], acc.at[dbuf[q]], add=True)

                @pl.when(ni < CHUNKS_PER_WORKER)
                def _():
                    gather_copy(ni, b).start()

            return _

        lax.fori_loop(0, CHUNKS_PER_WORKER // 4, group, None)
        plsc.subcore_barrier()

        pltpu.sync_copy(acc.at[pl.ds(r0, SLAB)],
                        out_hbm.at[pl.ds(c * N_ROWS + r0, SLAB)])

    k = pl.kernel(
        body,
        mesh=mesh,
        out_type=jax.ShapeDtypeStruct((NUM_CORES * N_ROWS, LATENT),
                                      jnp.float32),
        scratch_types=[
            pltpu.VMEM((EDGES_PER_WORKER,), jnp.int32),
            pltpu.VMEM((CHUNK,), jnp.int32),
            pltpu.VMEM((CHUNK,), jnp.int32),
            pltpu.VMEM((CHUNK,), jnp.int32),
            pltpu.VMEM((CHUNK,), jnp.int32),
            pltpu.VMEM((CHUNK, LATENT), jnp.float32),
            pltpu.VMEM((CHUNK, LATENT), jnp.float32),
            pltpu.VMEM_SHARED((N_ROWS, LATENT), jnp.float32),
            pltpu.SemaphoreType.DMA,
            pltpu.SemaphoreType.DMA,
            pltpu.SemaphoreType.DMA,
            pltpu.SemaphoreType.DMA,
            pltpu.SemaphoreType.DMA,
            pltpu.SemaphoreType.DMA,
        ],
    )
    return k(xs_ext, src1d, dst1d)


def _xw_body(z_ref, w_ref, dinv_ref, o_ref):
    o_ref[...] = dinv_ref[...] * jnp.dot(
        z_ref[...], w_ref[...], preferred_element_type=jnp.float32)


def _xw(z, W, dinv):
    n = z.shape[0]
    grid = (pl.cdiv(n, 1024),)
    return pl.pallas_call(
        _xw_body,
        grid=grid,
        in_specs=[
            pl.BlockSpec((1024, LATENT), lambda i: (i, 0)),
            pl.BlockSpec((LATENT, LATENT), lambda i: (0, 0)),
            pl.BlockSpec((1024, 1), lambda i: (i, 0)),
        ],
        out_specs=pl.BlockSpec((1024, LATENT), lambda i: (i, 0)),
        out_shape=jax.ShapeDtypeStruct((n, LATENT), jnp.float32),
    )(z, W, dinv[:, None])


def _h_body(p0_ref, p1_ref, xs_ref, dinv_ref, b_ref, o_ref):
    agg = p0_ref[...] + p1_ref[...] + xs_ref[...]
    h = jnp.maximum(dinv_ref[...] * agg + b_ref[...], 0.0)
    o_ref[...] = h.astype(jnp.bfloat16)


def _h_epilogue(part, xs, dinv, b):
    n = xs.shape[0]
    grid = (pl.cdiv(n, 1024),)
    nblk = N_ROWS // 1024
    return pl.pallas_call(
        _h_body,
        grid=grid,
        in_specs=[
            pl.BlockSpec((1024, LATENT), lambda i: (i, 0)),
            pl.BlockSpec((1024, LATENT), lambda i: (i + nblk, 0)),
            pl.BlockSpec((1024, LATENT), lambda i: (i, 0)),
            pl.BlockSpec((1024, 1), lambda i: (i, 0)),
            pl.BlockSpec((1, LATENT), lambda i: (0, 0)),
        ],
        out_specs=pl.BlockSpec((1024, LATENT), lambda i: (i, 0)),
        out_shape=jax.ShapeDtypeStruct((n, LATENT), jnp.bfloat16),
    )(part, part, xs, dinv[:, None], b[None, :])


def _decode_body(h_ref, g_ref, o_ref):
    o_ref[...] = jax.lax.dot_general(
        h_ref[...], g_ref[...],
        dimension_numbers=(((1,), (1,)), ((), ())),
        preferred_element_type=jnp.float32,
    )


def _decode(h):
    n = h.shape[0]
    assert h.dtype == jnp.bfloat16
    grid = (pl.cdiv(n, BM), pl.cdiv(n, BN))
    return pl.pallas_call(
        _decode_body,
        grid=grid,
        in_specs=[
            pl.BlockSpec((BM, LATENT), lambda i, j: (i, 0)),
            pl.BlockSpec((BN, LATENT), lambda i, j: (j, 0)),
        ],
        out_specs=pl.BlockSpec((BM, BN), lambda i, j: (i, j)),
        out_shape=jax.ShapeDtypeStruct((n, n), jnp.float32),
    )(h, h)


def kernel(z, edge_index, W, b):
    n = z.shape[0]
    src = edge_index[0].astype(jnp.int32)
    dst = edge_index[1].astype(jnp.int32)

    npad = E_PAD - N_EDGES
    pad_idx = jnp.arange(npad, dtype=jnp.int32)
    src1d = jnp.concatenate([src, pad_idx % N_NODES])
    dst1d = jnp.concatenate([dst, N_NODES + pad_idx % (N_ROWS - N_NODES)])
    dst2d = dst1d.reshape(E_PAD // CHUNK, CHUNK)

    deg_part = _sc_degree(dst2d)
    dinv = jax.lax.rsqrt(deg_part[:n] + deg_part[N_ROWS:N_ROWS + n] + 1.0)

    xs = dinv[:, None] * (z @ W)
    part = _sc_scatter(xs, src1d, dst1d)
    agg = part[:n] + part[N_ROWS:N_ROWS + n] + xs
    h = jax.nn.relu(dinv[:, None] * agg + b).astype(jnp.bfloat16)
    return _decode(h)


# speedup vs baseline: 1.0139x; 1.0139x over previous
import functools

import jax
import jax.numpy as jnp
from jax import lax
from jax.experimental import pallas as pl
from jax.experimental.pallas import tpu as pltpu
from jax.experimental.pallas import tpu_sc as plsc

N_NODES = 10000
N_EDGES = 320000
LATENT = 128

NUM_CORES = 2
NUM_SUBCORES = 16
NUM_WORKERS = NUM_CORES * NUM_SUBCORES
CHUNK = 128
N_CHUNKS = 2560
E_PAD = N_CHUNKS * CHUNK
N_ROWS = 10240
SLAB = N_ROWS // NUM_SUBCORES
CHUNKS_PER_WORKER = N_CHUNKS // NUM_WORKERS
EDGES_PER_WORKER = CHUNKS_PER_WORKER * CHUNK

BM = 2048
BN = 2048


def _deg_body(dst_hbm, out_hbm, didx, ones_v, zrow, acc, sem):
    c = lax.axis_index("c")
    s = lax.axis_index("s")

    def zinit(j, _):
        zrow[pl.ds(j * 16, 16)] = jnp.zeros((16,), jnp.float32)
        return _

    lax.fori_loop(0, SLAB // 16, zinit, None)
    for j in range(CHUNK // 16):
        ones_v[pl.ds(j * 16, 16)] = jnp.ones((16,), jnp.float32)
    pltpu.sync_copy(zrow, acc.at[pl.ds(s * SLAB, SLAB)])
    plsc.subcore_barrier()

    w = c * NUM_SUBCORES + s
    pltpu.sync_copy(dst_hbm.at[pl.ds(w * CHUNKS_PER_WORKER, CHUNKS_PER_WORKER)],
                    didx)
    copies = [
        pltpu.async_copy(ones_v, acc.at[didx.at[i]], sem, add=True)
        for i in range(CHUNKS_PER_WORKER)
    ]
    for cp in copies:
        cp.wait()
    plsc.subcore_barrier()

    r0 = s * SLAB
    pltpu.sync_copy(acc.at[pl.ds(r0, SLAB)],
                    out_hbm.at[pl.ds(c * N_ROWS + r0, SLAB)])


def _sc_degree(dst2d):
    mesh = plsc.VectorSubcoreMesh(core_axis_name="c", subcore_axis_name="s")
    k = pl.kernel(
        _deg_body,
        mesh=mesh,
        out_type=jax.ShapeDtypeStruct((NUM_CORES * N_ROWS,), jnp.float32),
        scratch_types=[
            pltpu.VMEM((CHUNKS_PER_WORKER, CHUNK), jnp.int32),
            pltpu.VMEM((CHUNK,), jnp.float32),
            pltpu.VMEM((SLAB,), jnp.float32),
            pltpu.VMEM_SHARED((N_ROWS,), jnp.float32),
            pltpu.SemaphoreType.DMA,
        ],
    )
    return k(dst2d)


def _sc_scatter(xs_ext, src1d, dst1d):
    mesh = plsc.VectorSubcoreMesh(core_axis_name="c", subcore_axis_name="s")

    def body(xs_hbm, src_hbm, dst_hbm, out_hbm,
             sidx, db0, db1, db2, db3, r0b, r1b, acc,
             i0, i1, i2, i3, g0, g1, psem):
        rows = (r0b, r1b)
        dbuf = (db0, db1, db2, db3)
        isem = (i0, i1, i2, i3)
        gsem = (g0, g1)
        c = lax.axis_index("c")
        s = lax.axis_index("s")

        w = c * NUM_SUBCORES + s
        e0 = w * EDGES_PER_WORKER

        def didx_fetch(ci, q):
            return pltpu.make_async_copy(
                dst_hbm.at[pl.ds(e0 + ci * CHUNK, CHUNK)], dbuf[q], isem[q])

        def gather_copy(ci, b):
            return pltpu.make_async_copy(
                xs_hbm.at[sidx.at[pl.ds(ci * CHUNK, CHUNK)]], rows[b],
                gsem[b])

        sidx_cp = pltpu.make_async_copy(
            src_hbm.at[pl.ds(e0, EDGES_PER_WORKER)], sidx, psem)
        sidx_cp.start()
        didx_fetch(0, 0).start()
        didx_fetch(1, 1).start()

        def zinit(j, _):
            for j16 in range(LATENT // 16):
                r0b[j, pl.ds(j16 * 16, 16)] = jnp.zeros((16,), jnp.float32)
            return _

        lax.fori_loop(0, CHUNK, zinit, None)
        r0 = s * SLAB
        zcp = [pltpu.make_async_copy(r0b, acc.at[pl.ds(r0 + t * CHUNK, CHUNK)],
                                     psem)
               for t in range(SLAB // CHUNK)]
        for cp in zcp:
            cp.start()
        for cp in zcp:
            cp.wait()
        sidx_cp.wait()
        plsc.subcore_barrier()

        gather_copy(0, 0).start()
        gather_copy(1, 1).start()

        def group(g, _):
            for j in range(4):
                ci = 4 * g + j
                b = j % 2
                q = j
                ni = ci + 2
                nq = (j + 2) % 4

                @pl.when(ni < CHUNKS_PER_WORKER)
                def _():
                    didx_fetch(ni, nq).start()

                gather_copy(ci, b).wait()
                didx_fetch(ci, q).wait()
                pltpu.sync_copy(rows[b], acc.at[dbuf[q]], add=True)

                @pl.when(ni < CHUNKS_PER_WORKER)
                def _():
                    gather_copy(ni, b).start()

            return _

        lax.fori_loop(0, CHUNKS_PER_WORKER // 4, group, None)
        plsc.subcore_barrier()

        pltpu.sync_copy(acc.at[pl.ds(r0, SLAB)],
                        out_hbm.at[pl.ds(c * N_ROWS + r0, SLAB)])

    k = pl.kernel(
        body,
        mesh=mesh,
        out_type=jax.ShapeDtypeStruct((NUM_CORES * N_ROWS, LATENT),
                                      jnp.float32),
        scratch_types=[
            pltpu.VMEM((EDGES_PER_WORKER,), jnp.int32),
            pltpu.VMEM((CHUNK,), jnp.int32),
            pltpu.VMEM((CHUNK,), jnp.int32),
            pltpu.VMEM((CHUNK,), jnp.int32),
            pltpu.VMEM((CHUNK,), jnp.int32),
            pltpu.VMEM((CHUNK, LATENT), jnp.float32),
            pltpu.VMEM((CHUNK, LATENT), jnp.float32),
            pltpu.VMEM_SHARED((N_ROWS, LATENT), jnp.float32),
            pltpu.SemaphoreType.DMA,
            pltpu.SemaphoreType.DMA,
            pltpu.SemaphoreType.DMA,
            pltpu.SemaphoreType.DMA,
            pltpu.SemaphoreType.DMA,
            pltpu.SemaphoreType.DMA,
            pltpu.SemaphoreType.DMA,
        ],
    )
    return k(xs_ext, src1d, dst1d)


def _xw_body(z_ref, w_ref, dinv_ref, o_ref):
    o_ref[...] = dinv_ref[...] * jnp.dot(
        z_ref[...], w_ref[...], preferred_element_type=jnp.float32)


def _xw(z, W, dinv):
    n = z.shape[0]
    grid = (pl.cdiv(n, 1024),)
    return pl.pallas_call(
        _xw_body,
        grid=grid,
        in_specs=[
            pl.BlockSpec((1024, LATENT), lambda i: (i, 0)),
            pl.BlockSpec((LATENT, LATENT), lambda i: (0, 0)),
            pl.BlockSpec((1024, 1), lambda i: (i, 0)),
        ],
        out_specs=pl.BlockSpec((1024, LATENT), lambda i: (i, 0)),
        out_shape=jax.ShapeDtypeStruct((n, LATENT), jnp.float32),
    )(z, W, dinv[:, None])


def _h_body(p0_ref, p1_ref, xs_ref, dinv_ref, b_ref, o_ref):
    agg = p0_ref[...] + p1_ref[...] + xs_ref[...]
    h = jnp.maximum(dinv_ref[...] * agg + b_ref[...], 0.0)
    o_ref[...] = h.astype(jnp.bfloat16)


def _h_epilogue(part, xs, dinv, b):
    n = xs.shape[0]
    grid = (pl.cdiv(n, 1024),)
    nblk = N_ROWS // 1024
    return pl.pallas_call(
        _h_body,
        grid=grid,
        in_specs=[
            pl.BlockSpec((1024, LATENT), lambda i: (i, 0)),
            pl.BlockSpec((1024, LATENT), lambda i: (i + nblk, 0)),
            pl.BlockSpec((1024, LATENT), lambda i: (i, 0)),
            pl.BlockSpec((1024, 1), lambda i: (i, 0)),
            pl.BlockSpec((1, LATENT), lambda i: (0, 0)),
        ],
        out_specs=pl.BlockSpec((1024, LATENT), lambda i: (i, 0)),
        out_shape=jax.ShapeDtypeStruct((n, LATENT), jnp.bfloat16),
    )(part, part, xs, dinv[:, None], b[None, :])


def _decode_body(h_ref, g_ref, o_ref):
    o_ref[...] = jax.lax.dot_general(
        h_ref[...], g_ref[...],
        dimension_numbers=(((1,), (1,)), ((), ())),
        preferred_element_type=jnp.float32,
    )


def _decode(h):
    n = h.shape[0]
    assert h.dtype == jnp.bfloat16
    grid = (pl.cdiv(n, BM), pl.cdiv(n, BN))
    return pl.pallas_call(
        _decode_body,
        grid=grid,
        in_specs=[
            pl.BlockSpec((BM, LATENT), lambda i, j: (i, 0)),
            pl.BlockSpec((BN, LATENT), lambda i, j: (j, 0)),
        ],
        out_specs=pl.BlockSpec((BM, BN), lambda i, j: (i, j)),
        out_shape=jax.ShapeDtypeStruct((n, n), jnp.float32),
    )(h, h)


def kernel(z, edge_index, W, b):
    n = z.shape[0]
    src = edge_index[0].astype(jnp.int32)
    dst = edge_index[1].astype(jnp.int32)

    npad = E_PAD - N_EDGES
    pad_idx = jnp.arange(npad, dtype=jnp.int32)
    src1d = jnp.concatenate([src, pad_idx % N_NODES])
    dst1d = jnp.concatenate([dst, N_NODES + pad_idx % (N_ROWS - N_NODES)])
    dst2d = dst1d.reshape(E_PAD // CHUNK, CHUNK)

    deg_part = _sc_degree(dst2d)
    dinv = jax.lax.rsqrt(deg_part[:n] + deg_part[N_ROWS:N_ROWS + n] + 1.0)

    xs = dinv[:, None] * (z @ W)
    part = _sc_scatter(xs, src1d, dst1d)
    agg = part[:n] + part[N_ROWS:N_ROWS + n] + xs
    h = jax.nn.relu(dinv[:, None] * agg + b).astype(jnp.bfloat16)
    return _decode(h)
